# Initial kernel scaffold; baseline (speedup 1.0000x reference)
import jax
import jax.numpy as jnp
from jax.experimental import pallas as pl


def kernel(input, edge_index, input_emb, W):
    raise NotImplementedError("write your pallas kernel here")


# speedup vs baseline: 6.8489x; 6.8489x over previous
import functools

import jax
import jax.numpy as jnp
from jax import lax
from jax.experimental import pallas as pl
from jax.experimental.pallas import tpu as pltpu
from jax.experimental.pallas import tpu_sc as plsc

ALPHA = 0.1
CHUNK = 128


def _sc_segment_partials(inp, src3d, dst3d):
    N, D = inp.shape
    NCH = src3d.shape[0]
    info = plsc.get_sparse_core_info()
    NC, NS = info.num_cores, info.num_subcores
    NW = NC * NS
    base_ch = NCH // NW
    rem_ch = NCH % NW
    rpt = ((N // NS) // 8) * 8
    extra = N - NS * rpt
    n_full = rpt // CHUNK
    n_rem = rpt % CHUNK

    mesh = plsc.VectorSubcoreMesh(core_axis_name="c", subcore_axis_name="s")

    @functools.partial(
        pl.kernel,
        out_type=jax.ShapeDtypeStruct((NC, N, D), jnp.float32),
        mesh=mesh,
        scratch_types=[
            pltpu.VMEM_SHARED((N, D), jnp.float32),
            pltpu.VMEM((CHUNK,), jnp.int32),
            pltpu.VMEM((CHUNK,), jnp.int32),
            pltpu.VMEM((CHUNK, D), jnp.float32),
            pltpu.SemaphoreType.DMA,
        ],
    )
    def k(inp_hbm, src_hbm, dst_hbm, out_hbm, acc, src_idx, dst_idx, rows, sem):
        c = lax.axis_index("c")
        s = lax.axis_index("s")
        w = s * NC + c

        zero = jnp.zeros((16,), jnp.float32)

        def zero_body(i, carry):
            for jcol in range(D // 16):
                rows[i, pl.ds(jcol * 16, 16)] = zero
            return carry

        lax.fori_loop(0, CHUNK, zero_body, 0)
        r0 = s * rpt
        for j in range(n_full):
            pltpu.sync_copy(rows, acc.at[pl.ds(r0 + j * CHUNK, CHUNK)])
        if n_rem:
            pltpu.sync_copy(
                rows.at[pl.ds(0, n_rem)],
                acc.at[pl.ds(r0 + n_full * CHUNK, n_rem)],
            )

        @pl.when(s == NS - 1)
        def _():
            if extra:
                pltpu.sync_copy(
                    rows.at[pl.ds(0, extra)],
                    acc.at[pl.ds(NS * rpt, extra)],
                )

        plsc.subcore_barrier()

        my_n = base_ch + jnp.where(w < rem_ch, 1, 0)
        my_base = w * base_ch + jnp.minimum(w, rem_ch)

        def edge_body(j, carry):
            ch = my_base + j
            pltpu.sync_copy(src_hbm.at[ch, 0], src_idx)
            pltpu.sync_copy(dst_hbm.at[ch, 0], dst_idx)
            pltpu.async_copy(inp_hbm.at[src_idx], rows, sem).wait()
            pltpu.sync_copy(rows, acc.at[dst_idx], add=True)
            return carry

        lax.fori_loop(0, my_n, edge_body, 0)
        plsc.subcore_barrier()

        for j in range(n_full):
            pltpu.sync_copy(acc.at[pl.ds(r0 + j * CHUNK, CHUNK)], rows)
            pltpu.sync_copy(rows, out_hbm.at[c, pl.ds(r0 + j * CHUNK, CHUNK)])
        if n_rem:
            pltpu.sync_copy(
                acc.at[pl.ds(r0 + n_full * CHUNK, n_rem)],
                rows.at[pl.ds(0, n_rem)],
            )
            pltpu.sync_copy(
                rows.at[pl.ds(0, n_rem)],
                out_hbm.at[c, pl.ds(r0 + n_full * CHUNK, n_rem)],
            )

        @pl.when(s == NS - 1)
        def _():
            if extra:
                pltpu.sync_copy(acc.at[pl.ds(NS * rpt, extra)], rows.at[pl.ds(0, extra)])
                pltpu.sync_copy(
                    rows.at[pl.ds(0, extra)],
                    out_hbm.at[c, pl.ds(NS * rpt, extra)],
                )

    return k(inp, src3d, dst3d)


def _blend(partials, input_emb):
    N, D = input_emb.shape
    BR = 1000
    grid = (N // BR,)

    def body(p0_ref, p1_ref, emb_ref, o_ref):
        o_ref[...] = (1.0 - ALPHA) * (p0_ref[...] + p1_ref[...]) + ALPHA * emb_ref[...]

    spec = pl.BlockSpec((BR, D), lambda i: (i, 0))
    return pl.pallas_call(
        body,
        grid=grid,
        in_specs=[spec, spec, spec],
        out_specs=spec,
        out_shape=jax.ShapeDtypeStruct((N, D), jnp.float32),
    )(partials[0], partials[1], input_emb)


def kernel(input, edge_index, input_emb, W):
    E = edge_index.shape[1]
    ei = edge_index.astype(jnp.int32)
    src3d = ei[1].reshape(E // CHUNK, 1, CHUNK)
    dst3d = ei[0].reshape(E // CHUNK, 1, CHUNK)
    partials = _sc_segment_partials(input, src3d, dst3d)
    return _blend(partials, input_emb)
